# baseline (speedup 1.0000x reference)
import functools

import jax
import jax.numpy as jnp
from jax import lax
from jax.experimental import pallas as pl
from jax.experimental.pallas import tpu as pltpu
from jax.experimental.pallas import tpu_sc as plsc

EMBED = 64
NUM_CORES = 2
NUM_SUBCORES = 16
NW = NUM_CORES * NUM_SUBCORES
SUB = 128
NSUB = 8
CHUNK = SUB * NSUB


def _gather_body(table_hbm, idx_hbm, out_hbm, idx_v, rows_v, sem):
    wid = lax.axis_index("s") * NUM_CORES + lax.axis_index("c")
    b_per_w = out_hbm.shape[0] // NW
    iters = b_per_w // CHUNK
    base = wid * b_per_w

    def body(i, carry):
        off = pl.multiple_of(base + i * CHUNK, CHUNK)
        row0 = pl.multiple_of(off // SUB, NSUB)
        pltpu.sync_copy(idx_hbm.at[pl.ds(row0, NSUB)], idx_v)
        copies = [
            pltpu.async_copy(
                table_hbm.at[idx_v.at[j]],
                rows_v.at[pl.ds(j * SUB, SUB)],
                sem,
            )
            for j in range(NSUB)
        ]
        for c in copies:
            c.wait()
        pltpu.sync_copy(rows_v, out_hbm.at[pl.ds(off, CHUNK)])
        return carry

    lax.fori_loop(0, iters, body, 0)


def kernel(sentence, word_embed_weight):
    batch, seq = sentence.shape
    B = batch * seq
    idx2d = sentence.reshape(B // SUB, SUB)
    mesh = plsc.VectorSubcoreMesh(core_axis_name="c", subcore_axis_name="s")
    run = pl.kernel(
        _gather_body,
        out_type=jax.ShapeDtypeStruct((B, EMBED), jnp.float32),
        mesh=mesh,
        scratch_types=[
            pltpu.VMEM((NSUB, SUB), jnp.int32),
            pltpu.VMEM((CHUNK, EMBED), jnp.float32),
            pltpu.SemaphoreType.DMA,
        ],
        compiler_params=pltpu.CompilerParams(use_tc_tiling_on_sc=False),
    )
    out = run(word_embed_weight, idx2d)
    return out.reshape(batch, seq, EMBED)


# speedup vs baseline: 1.0149x; 1.0149x over previous
import jax
import jax.numpy as jnp
from jax import lax
from jax.experimental import pallas as pl
from jax.experimental.pallas import tpu as pltpu
from jax.experimental.pallas import tpu_sc as plsc

EMBED = 64
NUM_CORES = 2
NUM_SUBCORES = 16
NW = NUM_CORES * NUM_SUBCORES
SUB = 128
NSUB = 4
CHUNK = SUB * NSUB
NBUF = 2


def _gather_body(table_hbm, idx_hbm, out_hbm, idx_v, rows_v,
                 sem_idx, sem_g, sem_out):
    wid = lax.axis_index("s") * NUM_CORES + lax.axis_index("c")
    b_per_w = out_hbm.shape[0] // NW
    iters = b_per_w // CHUNK
    base = wid * b_per_w
    base_row = base // SUB

    def idx_copy(i, p):
        row0 = pl.multiple_of(base_row + i * NSUB, NSUB)
        return pltpu.async_copy(
            idx_hbm.at[pl.ds(row0, NSUB)], idx_v.at[p], sem_idx)

    def out_copy(i, p):
        off = pl.multiple_of(base + i * CHUNK, CHUNK)
        return pltpu.async_copy(
            rows_v.at[p], out_hbm.at[pl.ds(off, CHUNK)], sem_out)

    idx_copy(0, 0)

    def body(i, carry):
        p = lax.rem(i, NBUF)

        @pl.when(i >= NBUF)
        def _():
            pltpu.make_async_copy(
                rows_v.at[p], out_hbm.at[pl.ds(0, CHUNK)], sem_out).wait()

        pltpu.make_async_copy(
            idx_hbm.at[pl.ds(0, NSUB)], idx_v.at[p], sem_idx).wait()

        gathers = [
            pltpu.async_copy(
                table_hbm.at[idx_v.at[p].at[j]],
                rows_v.at[p].at[pl.ds(j * SUB, SUB)],
                sem_g,
            )
            for j in range(NSUB)
        ]

        @pl.when(i + 1 < iters)
        def _():
            idx_copy(i + 1, 1 - p)

        for g in gathers:
            g.wait()

        out_copy(i, p)
        return carry

    lax.fori_loop(0, iters, body, 0)

    for _ in range(NBUF):
        pltpu.make_async_copy(
            rows_v.at[0], out_hbm.at[pl.ds(0, CHUNK)], sem_out).wait()


def kernel(sentence, word_embed_weight):
    batch, seq = sentence.shape
    B = batch * seq
    idx2d = sentence.reshape(B // SUB, SUB)
    mesh = plsc.VectorSubcoreMesh(core_axis_name="c", subcore_axis_name="s")
    run = pl.kernel(
        _gather_body,
        out_type=jax.ShapeDtypeStruct((B, EMBED), jnp.float32),
        mesh=mesh,
        scratch_types=[
            pltpu.VMEM((NBUF, NSUB, SUB), jnp.int32),
            pltpu.VMEM((NBUF, CHUNK, EMBED), jnp.float32),
            pltpu.SemaphoreType.DMA,
            pltpu.SemaphoreType.DMA,
            pltpu.SemaphoreType.DMA,
        ],
        compiler_params=pltpu.CompilerParams(use_tc_tiling_on_sc=False),
    )
    out = run(word_embed_weight, idx2d)
    return out.reshape(batch, seq, EMBED)


# speedup vs baseline: 1.0174x; 1.0024x over previous
import jax
import jax.numpy as jnp
from jax import lax
from jax.experimental import pallas as pl
from jax.experimental.pallas import tpu as pltpu
from jax.experimental.pallas import tpu_sc as plsc

EMBED = 64
NUM_CORES = 2
NUM_SUBCORES = 16
NW = NUM_CORES * NUM_SUBCORES
SUB = 512
NSUB = 1
CHUNK = SUB * NSUB
NBUF = 2


def _gather_body(table_hbm, idx_hbm, out_hbm, idx_v, rows_v,
                 sem_idx, sem_g, sem_out):
    wid = lax.axis_index("s") * NUM_CORES + lax.axis_index("c")
    b_per_w = out_hbm.shape[0] // NW
    iters = b_per_w // CHUNK
    base = wid * b_per_w
    base_row = base // SUB

    def idx_copy(i, p):
        row0 = pl.multiple_of(base_row + i * NSUB, NSUB)
        return pltpu.async_copy(
            idx_hbm.at[pl.ds(row0, NSUB)], idx_v.at[p], sem_idx)

    def out_copy(i, p):
        off = pl.multiple_of(base + i * CHUNK, CHUNK)
        return pltpu.async_copy(
            rows_v.at[p], out_hbm.at[pl.ds(off, CHUNK)], sem_out)

    idx_copy(0, 0)

    def body(i, carry):
        p = lax.rem(i, NBUF)

        @pl.when(i >= NBUF)
        def _():
            pltpu.make_async_copy(
                rows_v.at[p], out_hbm.at[pl.ds(0, CHUNK)], sem_out).wait()

        pltpu.make_async_copy(
            idx_hbm.at[pl.ds(0, NSUB)], idx_v.at[p], sem_idx).wait()

        gathers = [
            pltpu.async_copy(
                table_hbm.at[idx_v.at[p].at[j]],
                rows_v.at[p].at[pl.ds(j * SUB, SUB)],
                sem_g,
            )
            for j in range(NSUB)
        ]

        @pl.when(i + 1 < iters)
        def _():
            idx_copy(i + 1, 1 - p)

        for g in gathers:
            g.wait()

        out_copy(i, p)
        return carry

    lax.fori_loop(0, iters, body, 0)

    for _ in range(NBUF):
        pltpu.make_async_copy(
            rows_v.at[0], out_hbm.at[pl.ds(0, CHUNK)], sem_out).wait()


def kernel(sentence, word_embed_weight):
    batch, seq = sentence.shape
    B = batch * seq
    idx2d = sentence.reshape(B // SUB, SUB)
    mesh = plsc.VectorSubcoreMesh(core_axis_name="c", subcore_axis_name="s")
    run = pl.kernel(
        _gather_body,
        out_type=jax.ShapeDtypeStruct((B, EMBED), jnp.float32),
        mesh=mesh,
        scratch_types=[
            pltpu.VMEM((NBUF, NSUB, SUB), jnp.int32),
            pltpu.VMEM((NBUF, CHUNK, EMBED), jnp.float32),
            pltpu.SemaphoreType.DMA,
            pltpu.SemaphoreType.DMA,
            pltpu.SemaphoreType.DMA,
        ],
        compiler_params=pltpu.CompilerParams(use_tc_tiling_on_sc=False),
    )
    out = run(word_embed_weight, idx2d)
    return out.reshape(batch, seq, EMBED)
